# baseline (speedup 1.0000x reference)
import functools

import jax
import jax.numpy as jnp
from jax import lax
from jax.experimental import pallas as pl
from jax.experimental.pallas import tpu as pltpu

_NUM_CLASSES = 5
_C_PAD = 128
_C_SUB = 8


def _ce_kernel(x_ref, wt_ref, bt_ref, y_ref, out_ref, *, steps, total_rows):
    i = pl.program_id(0)
    j = pl.program_id(1)

    @pl.when(j == 0)
    def _():
        out_ref[0, 0, 0] = jnp.float32(0.0)

    lt = lax.dot_general(
        wt_ref[...],
        x_ref[...],
        dimension_numbers=(((1,), (1,)), ((), ())),
        preferred_element_type=jnp.float32,
    )
    ls = lt[0:_C_SUB, :] + bt_ref[...]

    m = jnp.max(ls, axis=0, keepdims=True)
    lse = m + jnp.log(jnp.sum(jnp.exp(ls - m), axis=0, keepdims=True))

    cls = lax.broadcasted_iota(jnp.int32, ls.shape, 0)
    y_row = y_ref[pl.ds(j * pl.num_programs(0) + i, 1), :]
    picked = jnp.sum(
        jnp.where(cls == y_row, ls, 0.0), axis=0, keepdims=True
    )

    out_ref[0, 0, 0] += jnp.sum(lse - picked)

    @pl.when(j == steps - 1)
    def _():
        out_ref[0, 0, 0] = out_ref[0, 0, 0] / jnp.float32(total_rows)


def _row_tile(rows):
    for t in (4096, 2048, 1024, 512, 256, 128, 64, 32, 16, 8):
        if rows % t == 0:
            return t
    return rows


def kernel(x, w_pad, b_pad, y):
    B, D = x.shape
    cores = 2 if B % 16 == 0 else 1
    rows_per_core = B // cores
    tm = _row_tile(rows_per_core)
    steps = rows_per_core // tm

    wt = w_pad.T
    bt = b_pad[0, :_C_SUB].reshape(_C_SUB, 1)
    y2 = y.reshape(B // tm, tm).astype(jnp.int32)

    partials = pl.pallas_call(
        functools.partial(_ce_kernel, steps=steps, total_rows=B),
        out_shape=jax.ShapeDtypeStruct((cores, 1, 1), jnp.float32),
        grid_spec=pltpu.PrefetchScalarGridSpec(
            num_scalar_prefetch=0,
            grid=(cores, steps),
            in_specs=[
                pl.BlockSpec((tm, D), lambda i, j, c=cores: (j * c + i, 0)),
                pl.BlockSpec((_C_PAD, D), lambda i, j: (0, 0)),
                pl.BlockSpec((_C_SUB, 1), lambda i, j: (0, 0)),
                pl.BlockSpec((B // tm, tm), lambda i, j: (0, 0)),
            ],
            out_specs=pl.BlockSpec(
                (1, 1, 1),
                lambda i, j: (i, 0, 0),
                memory_space=pltpu.MemorySpace.SMEM,
            ),
        ),
        compiler_params=pltpu.CompilerParams(
            dimension_semantics=("parallel", "arbitrary"),
        ),
    )(x, wt, bt, y2)

    return jnp.sum(partials)


# speedup vs baseline: 1.0445x; 1.0445x over previous
import functools

import jax
import jax.numpy as jnp
from jax import lax
from jax.experimental import pallas as pl
from jax.experimental.pallas import tpu as pltpu

_NUM_CLASSES = 5
_C_PAD = 128
_C_SUB = 8


def _ce_kernel(x_ref, wt_ref, bt_ref, y_ref, out_ref, *, steps, total_rows):
    j = pl.program_id(1)

    @pl.when(j == 0)
    def _():
        out_ref[0, 0, 0] = jnp.float32(0.0)

    lt = lax.dot_general(
        wt_ref[...],
        x_ref[...],
        dimension_numbers=(((0,), (1,)), ((), ())),
        preferred_element_type=jnp.float32,
    )
    ls = lt[0:_C_SUB, :] + bt_ref[...]

    m = jnp.max(ls, axis=0, keepdims=True)
    lse = m + jnp.log(jnp.sum(jnp.exp(ls - m), axis=0, keepdims=True))

    cls = lax.broadcasted_iota(jnp.int32, ls.shape, 0)
    picked = jnp.sum(
        jnp.where(cls == y_ref[...], ls, 0.0), axis=0, keepdims=True
    )

    out_ref[0, 0, 0] += jnp.sum(lse - picked)

    @pl.when(j == steps - 1)
    def _():
        out_ref[0, 0, 0] = out_ref[0, 0, 0] / jnp.float32(total_rows)


def _row_tile(rows):
    for t in (4096, 2048, 1024, 512, 256, 128, 64, 32, 16, 8):
        if rows % t == 0:
            return t
    return rows


def kernel(x, w_pad, b_pad, y):
    B, D = x.shape
    cores = 2 if B % 16 == 0 else 1
    rows_per_core = B // cores
    tm = _row_tile(rows_per_core)
    steps = rows_per_core // tm

    bt = b_pad[0, :_C_SUB].reshape(_C_SUB, 1)
    y2 = y.reshape(1, B).astype(jnp.int32)

    partials = pl.pallas_call(
        functools.partial(_ce_kernel, steps=steps, total_rows=B),
        out_shape=jax.ShapeDtypeStruct((cores, 1, 1), jnp.float32),
        grid_spec=pltpu.PrefetchScalarGridSpec(
            num_scalar_prefetch=0,
            grid=(cores, steps),
            in_specs=[
                pl.BlockSpec((tm, D), lambda i, j, c=cores: (j * c + i, 0)),
                pl.BlockSpec((D, _C_PAD), lambda i, j: (0, 0)),
                pl.BlockSpec((_C_SUB, 1), lambda i, j: (0, 0)),
                pl.BlockSpec((1, tm), lambda i, j, c=cores: (0, j * c + i)),
            ],
            out_specs=pl.BlockSpec(
                (1, 1, 1),
                lambda i, j: (i, 0, 0),
                memory_space=pltpu.MemorySpace.SMEM,
            ),
        ),
        compiler_params=pltpu.CompilerParams(
            dimension_semantics=("parallel", "arbitrary"),
        ),
    )(x, w_pad, bt, y2)

    return jnp.sum(partials)


# speedup vs baseline: 1.0606x; 1.0154x over previous
import functools

import jax
import jax.numpy as jnp
from jax import lax
from jax.experimental import pallas as pl
from jax.experimental.pallas import tpu as pltpu

_NUM_CLASSES = 5
_C_PAD = 128
_C_SUB = 8


def _ce_kernel(x_ref, wt_ref, bt_ref, y_ref, out_ref, *, steps, total_rows):
    j = pl.program_id(1)

    @pl.when(j == 0)
    def _():
        out_ref[0, 0, 0] = jnp.float32(0.0)

    lt = lax.dot_general(
        wt_ref[...],
        x_ref[...],
        dimension_numbers=(((0,), (1,)), ((), ())),
        preferred_element_type=jnp.float32,
    )
    bt = jnp.transpose(bt_ref[0:1, 0:_C_SUB])
    ls = lt[0:_C_SUB, :] + bt

    m = jnp.max(ls, axis=0, keepdims=True)
    lse = m + jnp.log(jnp.sum(jnp.exp(ls - m), axis=0, keepdims=True))

    cls = lax.broadcasted_iota(jnp.int32, ls.shape, 0)
    picked = jnp.sum(
        jnp.where(cls == y_ref[...], ls, 0.0), axis=0, keepdims=True
    )

    out_ref[0, 0, 0] += jnp.sum(lse - picked)

    @pl.when(j == steps - 1)
    def _():
        out_ref[0, 0, 0] = out_ref[0, 0, 0] / jnp.float32(total_rows)


def _row_tile(rows):
    for t in (4096, 2048, 1024, 512, 256, 128, 64, 32, 16, 8):
        if rows % t == 0:
            return t
    return rows


def kernel(x, w_pad, b_pad, y):
    B, D = x.shape
    cores = 2 if B % 16 == 0 else 1
    rows_per_core = B // cores
    tm = _row_tile(rows_per_core)
    steps = rows_per_core // tm

    y2 = y.reshape(1, B).astype(jnp.int32)

    partials = pl.pallas_call(
        functools.partial(_ce_kernel, steps=steps, total_rows=B),
        out_shape=jax.ShapeDtypeStruct((cores, 1, 1), jnp.float32),
        grid_spec=pltpu.PrefetchScalarGridSpec(
            num_scalar_prefetch=0,
            grid=(cores, steps),
            in_specs=[
                pl.BlockSpec((tm, D), lambda i, j, c=cores: (j * c + i, 0)),
                pl.BlockSpec((D, _C_PAD), lambda i, j: (0, 0)),
                pl.BlockSpec((1, _C_PAD), lambda i, j: (0, 0)),
                pl.BlockSpec((1, tm), lambda i, j, c=cores: (0, j * c + i)),
            ],
            out_specs=pl.BlockSpec(
                (1, 1, 1),
                lambda i, j: (i, 0, 0),
                memory_space=pltpu.MemorySpace.SMEM,
            ),
        ),
        compiler_params=pltpu.CompilerParams(
            dimension_semantics=("parallel", "arbitrary"),
        ),
    )(x, w_pad, b_pad, y2)

    return jnp.sum(partials)
